# baseline (device time: 261633 ns/iter reference)
import jax
import jax.numpy as jnp
from jax import lax
from jax.experimental import pallas as pl
from jax.experimental.pallas import tpu as pltpu

N_DEV = 8
M = 4096
K = 4096
N = 2048
K_PER = K // N_DEV
K_HALF = K // 2
M_CHUNK = 512
N_CHUNKS = M // M_CHUNK


def kernel(x, w_mat, scale_x, scale_w):
    m, k_per = x.shape
    _, n = w_mat.shape
    assert (m, k_per) == (M, K_PER) and n == N, (x.shape, w_mat.shape)

    x = x.astype(jnp.float8_e4m3fn)
    w_mat = w_mat.astype(jnp.float8_e5m2)

    def body(x_ref, w_ref, sx_ref, sw_ref, out_ref,
             x_comm, w_comm, acc, x_send, x_recv, w_send, w_recv, copy_sems):
        me = lax.axis_index("i")

        bar = pltpu.get_barrier_semaphore()
        for d in range(1, N_DEV):
            pl.semaphore_signal(
                bar, inc=1,
                device_id=((me + d) % N_DEV,),
                device_id_type=pl.DeviceIdType.MESH,
            )
        pl.semaphore_wait(bar, N_DEV - 1)

        def x_desc(d, dev):
            return pltpu.make_async_remote_copy(
                src_ref=x_ref,
                dst_ref=x_comm.at[:, pl.ds(d * K_PER, K_PER)],
                send_sem=x_send.at[d],
                recv_sem=x_recv.at[d],
                device_id=(dev,),
                device_id_type=pl.DeviceIdType.MESH,
            )

        def w_desc(d, dev):
            return pltpu.make_async_remote_copy(
                src_ref=w_ref,
                dst_ref=w_comm.at[pl.ds(d * K_PER, K_PER), :],
                send_sem=w_send.at[d],
                recv_sem=w_recv.at[d],
                device_id=(dev,),
                device_id_type=pl.DeviceIdType.MESH,
            )

        for d in range(1, N_DEV):
            x_desc(d, (me - d) % N_DEV).start()
            w_desc(d, (me - d) % N_DEV).start()

        x_comm[:, 0:K_PER] = x_ref[...]
        w_comm[0:K_PER, :] = w_ref[...]

        s = sx_ref[0] * sw_ref[0]

        def out_copy(c):
            return pltpu.make_async_copy(
                acc.at[c],
                out_ref.at[pl.ds(c * M_CHUNK, M_CHUNK), :],
                copy_sems.at[c],
            )

        for d in range(1, N_DEV // 2):
            x_desc(d, me).wait_recv()
            w_desc(d, me).wait_recv()
        for c in range(N_CHUNKS):
            acc[c] = lax.dot_general(
                x_comm[c * M_CHUNK:(c + 1) * M_CHUNK, 0:K_HALF],
                w_comm[0:K_HALF, :],
                dimension_numbers=(((1,), (0,)), ((), ())),
                preferred_element_type=jnp.float32,
            )

        for d in range(N_DEV // 2, N_DEV):
            x_desc(d, me).wait_recv()
            w_desc(d, me).wait_recv()
        for c in range(N_CHUNKS):
            acc[c] = (acc[c] + lax.dot_general(
                x_comm[c * M_CHUNK:(c + 1) * M_CHUNK, K_HALF:K],
                w_comm[K_HALF:K, :],
                dimension_numbers=(((1,), (0,)), ((), ())),
                preferred_element_type=jnp.float32,
            )) * s
            out_copy(c).start()

        for c in range(N_CHUNKS):
            out_copy(c).wait()

        for d in range(1, N_DEV):
            x_desc(d, (me - d) % N_DEV).wait_send()
            w_desc(d, (me - d) % N_DEV).wait_send()

    return pl.pallas_call(
        body,
        out_shape=jax.ShapeDtypeStruct((M, N), jnp.float32),
        in_specs=[
            pl.BlockSpec(memory_space=pltpu.VMEM),
            pl.BlockSpec(memory_space=pltpu.VMEM),
            pl.BlockSpec(memory_space=pltpu.SMEM),
            pl.BlockSpec(memory_space=pltpu.SMEM),
        ],
        out_specs=pl.BlockSpec(memory_space=pl.ANY),
        scratch_shapes=[
            pltpu.VMEM((M, K), x.dtype),
            pltpu.VMEM((K, N), w_mat.dtype),
            pltpu.VMEM((N_CHUNKS, M_CHUNK, N), jnp.float32),
            pltpu.SemaphoreType.DMA((N_DEV,)),
            pltpu.SemaphoreType.DMA((N_DEV,)),
            pltpu.SemaphoreType.DMA((N_DEV,)),
            pltpu.SemaphoreType.DMA((N_DEV,)),
            pltpu.SemaphoreType.DMA((N_CHUNKS,)),
        ],
        compiler_params=pltpu.CompilerParams(
            collective_id=0,
            vmem_limit_bytes=128 * 1024 * 1024,
        ),
    )(x, w_mat, scale_x, scale_w)


# device time: 233574 ns/iter; 1.1201x vs baseline; 1.1201x over previous
import functools

import jax
import jax.numpy as jnp
from jax import lax
from jax.experimental import pallas as pl
from jax.experimental.pallas import tpu as pltpu

N_DEV = 8
M = 4096
K = 4096
N = 2048
K_PER = K // N_DEV
M_CHUNK = 1024


def kernel(x, w_mat, scale_x, scale_w):
    m, k_per = x.shape
    _, n = w_mat.shape
    assert (m, k_per) == (M, K_PER) and n == N, (x.shape, w_mat.shape)

    x = x.astype(jnp.float8_e4m3fn)
    w_mat = w_mat.astype(jnp.float8_e5m2)

    def body(x_ref, w_ref, sx_ref, sw_ref, out_ref,
             x_comm, w_comm, acc, x_send, x_recv, w_send, w_recv, copy_sems):
        me = lax.axis_index("i")

        bar = pltpu.get_barrier_semaphore()
        for d in range(1, N_DEV):
            pl.semaphore_signal(
                bar, inc=1,
                device_id=((me + d) % N_DEV,),
                device_id_type=pl.DeviceIdType.MESH,
            )
        pl.semaphore_wait(bar, N_DEV - 1)

        def x_desc(j, p):
            return pltpu.make_async_remote_copy(
                src_ref=x_ref,
                dst_ref=x_comm.at[:, pl.ds(j * K_PER, K_PER)],
                send_sem=x_send.at[p],
                recv_sem=x_recv.at[j],
                device_id=(p,),
                device_id_type=pl.DeviceIdType.MESH,
            )

        def w_desc(j, p):
            return pltpu.make_async_remote_copy(
                src_ref=w_ref,
                dst_ref=w_comm.at[pl.ds(j * K_PER, K_PER), :],
                send_sem=w_send.at[p],
                recv_sem=w_recv.at[j],
                device_id=(p,),
                device_id_type=pl.DeviceIdType.MESH,
            )

        for j in range(N_DEV):
            @pl.when(me == j)
            def _(j=j):
                for p in range(N_DEV):
                    if p != j:
                        x_desc(j, p).start()
                        w_desc(j, p).start()
                x_comm[:, j * K_PER:(j + 1) * K_PER] = x_ref[...]
                w_comm[j * K_PER:(j + 1) * K_PER, :] = w_ref[...]

        for j in range(N_DEV):
            @pl.when(me != j)
            def _(j=j):
                x_desc(j, j).wait_recv()
                w_desc(j, j).wait_recv()

        s = sx_ref[0] * sw_ref[0]
        n_chunks = M // M_CHUNK

        def out_copy(c):
            return pltpu.make_async_copy(
                acc.at[c % 2],
                out_ref.at[pl.ds(c * M_CHUNK, M_CHUNK), :],
                copy_sems.at[c % 2],
            )

        for c in range(n_chunks):
            if c >= 2:
                out_copy(c - 2).wait()
            acc[c % 2] = lax.dot_general(
                x_comm[c * M_CHUNK:(c + 1) * M_CHUNK, :], w_comm[...],
                dimension_numbers=(((1,), (0,)), ((), ())),
                preferred_element_type=jnp.float32,
            ) * s
            out_copy(c).start()
        for c in range(n_chunks - 2, n_chunks):
            out_copy(c).wait()

        for j in range(N_DEV):
            @pl.when(me == j)
            def _(j=j):
                for p in range(N_DEV):
                    if p != j:
                        x_desc(j, p).wait_send()
                        w_desc(j, p).wait_send()

    return pl.pallas_call(
        body,
        out_shape=jax.ShapeDtypeStruct((M, N), jnp.float32),
        in_specs=[
            pl.BlockSpec(memory_space=pltpu.VMEM),
            pl.BlockSpec(memory_space=pltpu.VMEM),
            pl.BlockSpec(memory_space=pltpu.SMEM),
            pl.BlockSpec(memory_space=pltpu.SMEM),
        ],
        out_specs=pl.BlockSpec(memory_space=pl.ANY),
        scratch_shapes=[
            pltpu.VMEM((M, K), x.dtype),
            pltpu.VMEM((K, N), w_mat.dtype),
            pltpu.VMEM((2, M_CHUNK, N), jnp.float32),
            pltpu.SemaphoreType.DMA((N_DEV,)),
            pltpu.SemaphoreType.DMA((N_DEV,)),
            pltpu.SemaphoreType.DMA((N_DEV,)),
            pltpu.SemaphoreType.DMA((N_DEV,)),
            pltpu.SemaphoreType.DMA((2,)),
        ],
        compiler_params=pltpu.CompilerParams(
            collective_id=0,
            vmem_limit_bytes=128 * 1024 * 1024,
        ),
    )(x, w_mat, scale_x, scale_w)
